# baseline (speedup 1.0000x reference)
import dataclasses
import math

import numpy as np

import jax
import jax.numpy as jnp
from jax.experimental import pallas as pl
from jax.experimental.pallas import tpu as pltpu
from jax.experimental.pallas import tpu_sc as plsc

N = 4096
V = 32000
LS = 0.1
PAD = 0

_EPS = float(np.float32(LS / (V - 1)))
_P = 1.0 - LS
_C1 = _EPS * math.log(_EPS)
_C2 = _P * math.log(_P)

_ROW_BLK = 128
_GATHER_W = 128


def _colsum_body(xa_ref, xb_ref, y_ref, cs_ref, g_ref):
    i = pl.program_id(0)

    @pl.when(i == 0)
    def _init():
        cs_ref[...] = jnp.zeros_like(cs_ref)

    xa = xa_ref[...]
    xb = xb_ref[...]
    cs_ref[...] += (jnp.sum(xa, axis=0, keepdims=True)
                    + jnp.sum(xb, axis=0, keepdims=True))
    half = _ROW_BLK // 2
    cols = jax.lax.broadcasted_iota(jnp.int32, (half, V), 1)
    ya = y_ref[pl.ds(i * _ROW_BLK, half), :]
    yb = y_ref[pl.ds(i * _ROW_BLK + half, half), :]
    g_ref[pl.ds(i * _ROW_BLK, half), :] = jnp.sum(
        jnp.where(cols == ya, xa, jnp.float32(0.0)), axis=1, keepdims=True)
    g_ref[pl.ds(i * _ROW_BLK + half, half), :] = jnp.sum(
        jnp.where(cols == yb, xb, jnp.float32(0.0)), axis=1, keepdims=True)


def _combine_body(cs_ref, y_ref, g_ref, yy_ref, o_ref):
    yv = y_ref[...]
    cs = cs_ref[...]
    gv = g_ref[...]
    yyv = yy_ref[...]
    word = jnp.sum((yv != PAD).astype(jnp.float32))
    m = jnp.float32(N) - word
    masked = (yv < N) & (yyv == PAD)
    K = jnp.float32(N) - jnp.sum(masked.astype(jnp.float32))
    G = jnp.sum(jnp.where(masked, jnp.float32(0.0), gv))
    S_total = jnp.sum(cs)
    S_masked = jnp.sum(jnp.where(yv == PAD, cs[0:32, :], jnp.float32(0.0)))
    kl = (jnp.float32(N) * (jnp.float32(V) - m) * jnp.float32(_C1)
          + K * jnp.float32(_C2 - _C1)
          - jnp.float32(_EPS) * (S_total - S_masked)
          - jnp.float32(_P - _EPS) * G)
    o_ref[...] = (kl / word)[None, None]


def _sc_gather_yy(y_tbl, y_rows):
    mesh = plsc.VectorSubcoreMesh(core_axis_name="c", subcore_axis_name="s")
    cp = pltpu.CompilerParams()
    if "needs_layout_passes" in pltpu.CompilerParams.__dataclass_fields__:
        cp = dataclasses.replace(cp, needs_layout_passes=False)

    @pl.kernel(
        out_type=jax.ShapeDtypeStruct((N // _GATHER_W, _GATHER_W), jnp.int32),
        mesh=mesh,
        compiler_params=cp,
    )
    def run(ytbl_hbm, yrows_hbm, yy_hbm):
        def body(ytbl_vmem, yc_vmem, yy_vmem):
            @pl.loop(0, _GATHER_W, step=16)
            def _(k):
                idx = jnp.minimum(yc_vmem[0, pl.ds(k, 16)], N - 1)
                vals = plsc.load_gather(ytbl_vmem, [jnp.zeros_like(idx), idx])
                yy_vmem[0, pl.ds(k, 16)] = vals

        pltpu.emit_pipeline(
            body,
            grid=(N // _GATHER_W,),
            in_specs=[
                pl.BlockSpec((1, N), lambda i: (0, 0)),
                pl.BlockSpec((1, _GATHER_W), lambda i: (0, i)),
            ],
            out_specs=[pl.BlockSpec((1, _GATHER_W), lambda i: (i, 0))],
            core_axis_name=("c", "s"),
            dimension_semantics=(pltpu.PARALLEL,),
        )(ytbl_hbm, yrows_hbm, yy_hbm)

    return run(y_tbl, y_rows)


def kernel(out, y):
    y = y.reshape(-1).astype(jnp.int32)
    out2 = out.reshape(N, V)

    yy = _sc_gather_yy(y.reshape(1, N), y.reshape(1, N))

    colsum, g = pl.pallas_call(
        _colsum_body,
        grid=(N // _ROW_BLK,),
        in_specs=[
            pl.BlockSpec((_ROW_BLK // 2, V), lambda i: (2 * i, 0)),
            pl.BlockSpec((_ROW_BLK // 2, V), lambda i: (2 * i + 1, 0)),
            pl.BlockSpec((N, 1), lambda i: (0, 0)),
        ],
        out_specs=[
            pl.BlockSpec((1, V), lambda i: (0, 0)),
            pl.BlockSpec((N, 1), lambda i: (0, 0)),
        ],
        out_shape=[
            jax.ShapeDtypeStruct((1, V), jnp.float32),
            jax.ShapeDtypeStruct((N, 1), jnp.float32),
        ],
    )(out2, out2, y.reshape(N, 1))

    res = pl.pallas_call(
        _combine_body,
        in_specs=[
            pl.BlockSpec((V // 128, 128), lambda: (0, 0)),
            pl.BlockSpec((N // 128, 128), lambda: (0, 0)),
            pl.BlockSpec((N // 128, 128), lambda: (0, 0)),
            pl.BlockSpec((N // 128, 128), lambda: (0, 0)),
        ],
        out_specs=pl.BlockSpec((1, 1), lambda: (0, 0)),
        out_shape=jax.ShapeDtypeStruct((1, 1), jnp.float32),
    )(colsum.reshape(V // 128, 128), y.reshape(N // 128, 128),
      g.reshape(N // 128, 128), yy)

    return res[0, 0]


# speedup vs baseline: 1.0136x; 1.0136x over previous
import dataclasses
import math

import numpy as np

import jax
import jax.numpy as jnp
from jax.experimental import pallas as pl
from jax.experimental.pallas import tpu as pltpu
from jax.experimental.pallas import tpu_sc as plsc

N = 4096
V = 32000
LS = 0.1
PAD = 0

_EPS = float(np.float32(LS / (V - 1)))
_P = 1.0 - LS
_C1 = _EPS * math.log(_EPS)
_C2 = _P * math.log(_P)

_ROW_BLK = 128
_GATHER_W = 128


def _colsum_body(x_ref, y_ref, cs_ref, g_ref):
    i = pl.program_id(0)

    @pl.when(i == 0)
    def _init():
        cs_ref[...] = jnp.zeros_like(cs_ref)

    x = x_ref[...]
    part = x[0:8]
    for k in range(1, _ROW_BLK // 8):
        part = part + x[8 * k:8 * (k + 1)]
    cs_ref[...] += part
    cols = jax.lax.broadcasted_iota(jnp.int32, (_ROW_BLK, V), 1)
    yv = y_ref[pl.ds(i * _ROW_BLK, _ROW_BLK), :]
    g_ref[pl.ds(i * _ROW_BLK, _ROW_BLK), :] = jnp.sum(
        jnp.where(cols == yv, x, jnp.float32(0.0)),
        axis=1, keepdims=True)


def _combine_body(cs_ref, y_ref, yy_ref, g_ref, o_ref):
    yv = y_ref[...]
    yyv = yy_ref[...]
    cs8 = cs_ref[...]
    word = jnp.sum((yv != PAD).astype(jnp.float32))
    m = jnp.float32(N) - word
    masked = (yv < N) & (yyv == PAD)
    K = jnp.float32(N) - jnp.sum(masked.astype(jnp.float32))
    u = jnp.where(masked, jnp.float32(0.0), jnp.float32(1.0))
    G = jax.lax.dot_general(u, g_ref[...], (((1,), (0,)), ((), ())),
                            preferred_element_type=jnp.float32)[0, 0]
    S_total = jnp.sum(cs8)
    cs_first = jnp.sum(cs8[:, 0:N], axis=0, keepdims=True)
    S_masked = jnp.sum(jnp.where(yv == PAD, cs_first, jnp.float32(0.0)))
    kl = (jnp.float32(N) * (jnp.float32(V) - m) * jnp.float32(_C1)
          + K * jnp.float32(_C2 - _C1)
          - jnp.float32(_EPS) * (S_total - S_masked)
          - jnp.float32(_P - _EPS) * G)
    o_ref[...] = (kl / word)[None, None]


def _sc_gather_yy(y_tbl, y_idx):
    mesh = plsc.VectorSubcoreMesh(core_axis_name="c", subcore_axis_name="s")
    cp = pltpu.CompilerParams()
    if "needs_layout_passes" in pltpu.CompilerParams.__dataclass_fields__:
        cp = dataclasses.replace(cp, needs_layout_passes=False)

    @pl.kernel(
        out_type=jax.ShapeDtypeStruct((1, N), jnp.int32),
        mesh=mesh,
        compiler_params=cp,
    )
    def run(ytbl_hbm, yidx_hbm, yy_hbm):
        def body(ytbl_vmem, yc_vmem, yy_vmem):
            @pl.loop(0, _GATHER_W, step=16)
            def _(k):
                idx = jnp.minimum(yc_vmem[0, pl.ds(k, 16)], N - 1)
                vals = plsc.load_gather(ytbl_vmem, [jnp.zeros_like(idx), idx])
                yy_vmem[0, pl.ds(k, 16)] = vals

        pltpu.emit_pipeline(
            body,
            grid=(N // _GATHER_W,),
            in_specs=[
                pl.BlockSpec((1, N), lambda i: (0, 0)),
                pl.BlockSpec((1, _GATHER_W), lambda i: (0, i)),
            ],
            out_specs=[pl.BlockSpec((1, _GATHER_W), lambda i: (0, i))],
            core_axis_name=("c", "s"),
            dimension_semantics=(pltpu.PARALLEL,),
        )(ytbl_hbm, yidx_hbm, yy_hbm)

    return run(y_tbl, y_idx)


def kernel(out, y):
    y = y.reshape(-1).astype(jnp.int32)
    out2 = out.reshape(N, V)
    y_row = y.reshape(1, N)

    yy = _sc_gather_yy(y_row, y_row)

    cs8, g = pl.pallas_call(
        _colsum_body,
        grid=(N // _ROW_BLK,),
        in_specs=[
            pl.BlockSpec((_ROW_BLK, V), lambda i: (i, 0)),
            pl.BlockSpec((N, 1), lambda i: (0, 0)),
        ],
        out_specs=[
            pl.BlockSpec((8, V), lambda i: (0, 0)),
            pl.BlockSpec((N, 1), lambda i: (0, 0)),
        ],
        out_shape=[
            jax.ShapeDtypeStruct((8, V), jnp.float32),
            jax.ShapeDtypeStruct((N, 1), jnp.float32),
        ],
    )(out2, y.reshape(N, 1))

    res = pl.pallas_call(
        _combine_body,
        in_specs=[
            pl.BlockSpec((8, V), lambda: (0, 0)),
            pl.BlockSpec((1, N), lambda: (0, 0)),
            pl.BlockSpec((1, N), lambda: (0, 0)),
            pl.BlockSpec((N, 1), lambda: (0, 0)),
        ],
        out_specs=pl.BlockSpec((1, 1), lambda: (0, 0)),
        out_shape=jax.ShapeDtypeStruct((1, 1), jnp.float32),
    )(cs8, y_row, yy, g)

    return res[0, 0]


# speedup vs baseline: 1.0194x; 1.0057x over previous
import dataclasses
import math

import numpy as np

import jax
import jax.numpy as jnp
from jax.experimental import pallas as pl
from jax.experimental.pallas import tpu as pltpu
from jax.experimental.pallas import tpu_sc as plsc

N = 4096
V = 32000
LS = 0.1
PAD = 0

_EPS = float(np.float32(LS / (V - 1)))
_P = 1.0 - LS
_C1 = _EPS * math.log(_EPS)
_C2 = _P * math.log(_P)

_ROW_BLK = 128
_GATHER_W = 128


def _colsum_body(x_ref, y_ref, cs_ref, g_ref):
    i = pl.program_id(0)

    @pl.when(i == 0)
    def _init():
        cs_ref[...] = jnp.zeros_like(cs_ref)

    x = x_ref[...]
    parts = [x[8 * k:8 * (k + 1)] for k in range(_ROW_BLK // 8)]
    while len(parts) > 1:
        parts = [parts[j] + parts[j + 1] for j in range(0, len(parts), 2)]
    cs_ref[...] += parts[0]
    cols = jax.lax.broadcasted_iota(jnp.int32, (_ROW_BLK, V), 1)
    yv = jnp.transpose(y_ref[0:1, pl.ds(i * _ROW_BLK, _ROW_BLK)])
    g_ref[pl.ds(i * _ROW_BLK, _ROW_BLK), :] = jnp.sum(
        jnp.where(cols == yv, x, jnp.float32(0.0)),
        axis=1, keepdims=True)


def _combine_body(cs_ref, y_ref, yy_ref, g_ref, o_ref):
    yv = y_ref[...]
    yyv = yy_ref[...]
    cs8 = cs_ref[...]
    word = jnp.sum((yv != PAD).astype(jnp.float32))
    m = jnp.float32(N) - word
    masked = (yv < N) & (yyv == PAD)
    K = jnp.float32(N) - jnp.sum(masked.astype(jnp.float32))
    u = jnp.where(masked, jnp.float32(0.0), jnp.float32(1.0))
    G = jax.lax.dot_general(u, g_ref[...], (((1,), (0,)), ((), ())),
                            preferred_element_type=jnp.float32)[0, 0]
    S_total = jnp.sum(cs8)
    cs_first = jnp.sum(cs8[:, 0:N], axis=0, keepdims=True)
    S_masked = jnp.sum(jnp.where(yv == PAD, cs_first, jnp.float32(0.0)))
    kl = (jnp.float32(N) * (jnp.float32(V) - m) * jnp.float32(_C1)
          + K * jnp.float32(_C2 - _C1)
          - jnp.float32(_EPS) * (S_total - S_masked)
          - jnp.float32(_P - _EPS) * G)
    o_ref[...] = (kl / word)[None, None]


def _sc_gather_yy(y_tbl, y_idx):
    mesh = plsc.VectorSubcoreMesh(core_axis_name="c", subcore_axis_name="s")
    cp = pltpu.CompilerParams()
    if "needs_layout_passes" in pltpu.CompilerParams.__dataclass_fields__:
        cp = dataclasses.replace(cp, needs_layout_passes=False)

    @pl.kernel(
        out_type=jax.ShapeDtypeStruct((1, N), jnp.int32),
        mesh=mesh,
        compiler_params=cp,
    )
    def run(ytbl_hbm, yidx_hbm, yy_hbm):
        def body(ytbl_vmem, yc_vmem, yy_vmem):
            @pl.loop(0, _GATHER_W, step=16)
            def _(k):
                idx = jnp.minimum(yc_vmem[0, pl.ds(k, 16)], N - 1)
                vals = plsc.load_gather(ytbl_vmem, [jnp.zeros_like(idx), idx])
                yy_vmem[0, pl.ds(k, 16)] = vals

        pltpu.emit_pipeline(
            body,
            grid=(N // _GATHER_W,),
            in_specs=[
                pl.BlockSpec((1, N), lambda i: (0, 0)),
                pl.BlockSpec((1, _GATHER_W), lambda i: (0, i)),
            ],
            out_specs=[pl.BlockSpec((1, _GATHER_W), lambda i: (0, i))],
            core_axis_name=("c", "s"),
            dimension_semantics=(pltpu.PARALLEL,),
        )(ytbl_hbm, yidx_hbm, yy_hbm)

    return run(y_tbl, y_idx)


def kernel(out, y):
    y = y.reshape(-1).astype(jnp.int32)
    out2 = out.reshape(N, V)
    y_row = y.reshape(1, N)

    yy = _sc_gather_yy(y_row, y_row)

    cs8, g = pl.pallas_call(
        _colsum_body,
        grid=(N // _ROW_BLK,),
        in_specs=[
            pl.BlockSpec((_ROW_BLK, V), lambda i: (i, 0)),
            pl.BlockSpec((1, N), lambda i: (0, 0)),
        ],
        out_specs=[
            pl.BlockSpec((8, V), lambda i: (0, 0)),
            pl.BlockSpec((N, 1), lambda i: (0, 0)),
        ],
        out_shape=[
            jax.ShapeDtypeStruct((8, V), jnp.float32),
            jax.ShapeDtypeStruct((N, 1), jnp.float32),
        ],
    )(out2, y_row)

    res = pl.pallas_call(
        _combine_body,
        in_specs=[
            pl.BlockSpec((8, V), lambda: (0, 0)),
            pl.BlockSpec((1, N), lambda: (0, 0)),
            pl.BlockSpec((1, N), lambda: (0, 0)),
            pl.BlockSpec((N, 1), lambda: (0, 0)),
        ],
        out_specs=pl.BlockSpec((1, 1), lambda: (0, 0)),
        out_shape=jax.ShapeDtypeStruct((1, 1), jnp.float32),
    )(cs8, y_row, yy, g)

    return res[0, 0]
